# baseline (speedup 1.0000x reference)
import jax
import jax.numpy as jnp
from jax.experimental import pallas as pl
from jax.experimental.pallas import tpu as pltpu

_N = 32768
_B = 16
_H = 2048
_NHALF = 4
_BLK = _H * _NHALF
_NB = _N // _BLK
_HI = jax.lax.Precision.HIGHEST

_C00 = (((0,), (0,)), ((), ()))
_C11 = (((1,), (1,)), ((), ()))


def _body(cu_ref, x_ref, mean_c_ref, inv_c_ref, sc_ref, ec_ref, sr_ref, er_ref,
          w0_ref, b0c_ref, w1_ref, b1c_ref, w2_ref, b2c_ref,
          w4_ref, b4r_ref, w5_ref, b5r_ref, wout_ref, boutr_ref,
          out_ref, vbuf, segsum, tm, psum, pmn, pmx):
    p = pl.program_id(0)
    i = pl.program_id(1)
    relu = jax.nn.relu

    insegs, onehots = [], []
    for h in range(_NHALF):
        idx = (jax.lax.broadcasted_iota(jnp.int32, (_B, _H), 1)
               + i * _BLK + h * _H)
        inseg = (idx >= sc_ref[...]) & (idx < ec_ref[...])
        insegs.append(inseg)
        onehots.append(inseg.astype(jnp.float32))

    @pl.when(jnp.logical_and(p == 0, i == 0))
    def _init0():
        segsum[...] = jnp.zeros_like(segsum)

    @pl.when(p == 0)
    def _pass0():
        acc = segsum[...]
        for h in range(_NHALF):
            x = ((x_ref[:, h * _H:(h + 1) * _H] - mean_c_ref[...])
                 * inv_c_ref[...])
            v = relu(jax.lax.dot_general(w0_ref[...], x, _C00)
                     + b0c_ref[...])
            vbuf[:, pl.ds(i * _BLK + h * _H, _H)] = v
            acc += jax.lax.dot_general(v, onehots[h], _C11, precision=_HI)
        segsum[...] = acc

    @pl.when(jnp.logical_and(p == 1, i == 0))
    def _init1():
        cnt = jnp.maximum(er_ref[...] - sr_ref[...], 1).astype(jnp.float32)
        gmean_t = segsum[...] / cnt
        tm[...] = (jax.lax.dot_general(w1_ref[32:64, :], gmean_t, _C00)
                   + b1c_ref[...])
        psum[...] = jnp.zeros_like(psum)
        pmn[...] = jnp.full_like(pmn, jnp.inf)
        pmx[...] = jnp.full_like(pmx, -jnp.inf)

    @pl.when(p == 1)
    def _pass1():
        v3s = []
        for h in range(_NHALF):
            v = vbuf[:, pl.ds(i * _BLK + h * _H, _H)]
            v2 = relu(jax.lax.dot_general(w1_ref[0:32, :], v, _C00)
                      + jnp.dot(tm[...], onehots[h], precision=_HI))
            v3s.append(relu(jax.lax.dot_general(w2_ref[...], v2, _C00)
                            + b2c_ref[...]))
        acc = psum[...]
        for h in range(_NHALF):
            acc += jax.lax.dot_general(v3s[h], onehots[h], _C11, precision=_HI)
        psum[...] = acc
        for h in range(_NHALF):
            base = i * _BLK + h * _H
            for s in range(_B):
                lo = cu_ref[s]
                hi = cu_ref[s + 1]

                @pl.when(jnp.logical_and(hi > base, lo < base + _H))
                def _minmax(s=s, h=h):
                    m = insegs[h][s:s + 1, :]
                    mn = jnp.min(jnp.where(m, v3s[h], jnp.inf),
                                 axis=1, keepdims=True)
                    mx = jnp.max(jnp.where(m, v3s[h], -jnp.inf),
                                 axis=1, keepdims=True)
                    pmn[:, s:s + 1] = jnp.minimum(pmn[:, s:s + 1], mn)
                    pmx[:, s:s + 1] = jnp.maximum(pmx[:, s:s + 1], mx)

    @pl.when(jnp.logical_and(p == 1, i == _NB - 1))
    def _final():
        cntd = er_ref[...] - sr_ref[...]
        cnt = jnp.maximum(cntd, 1).astype(jnp.float32)
        valid = cntd > 0
        pmean_t = psum[...] / cnt
        pmn_t = jnp.where(valid, pmn[...], 0.0)
        pmx_t = jnp.where(valid, pmx[...], 0.0)
        h_t = jnp.concatenate([pmean_t, pmn_t, pmx_t], axis=0)
        h1 = relu(jax.lax.dot_general(h_t, w4_ref[...], _C00)
                  + b4r_ref[...])
        h2 = relu(jnp.dot(h1, w5_ref[...]) + b5r_ref[...])
        out_ref[...] = jnp.dot(h2, wout_ref[...]) + boutr_ref[...]


def kernel(flat, norm_mean, norm_var, W0, b0, W1, b1, W2, b2, W4, b4, W5, b5,
           Wout, bout, cu_seqlens):
    inv = 1.0 / jnp.sqrt(norm_var)
    x_t = flat.T
    starts = cu_seqlens[:-1]
    ends = cu_seqlens[1:]

    def full(shape):
        return pl.BlockSpec(shape, lambda p, i: (0, 0))

    out = pl.pallas_call(
        _body,
        grid=(2, _NB),
        in_specs=[
            pl.BlockSpec(memory_space=pltpu.SMEM),
            pl.BlockSpec((4, _BLK),
                         lambda p, i: (0, jnp.where(p == 0, i, _NB - 1))),
            full((4, 1)), full((4, 1)),
            full((_B, 1)), full((_B, 1)), full((1, _B)), full((1, _B)),
            full((4, 32)), full((32, 1)), full((64, 32)), full((32, 1)),
            full((32, 32)), full((32, 1)),
            full((96, 128)), full((1, 128)), full((128, 128)), full((1, 128)),
            full((128, 5)), full((1, 5)),
        ],
        out_specs=full((_B, 5)),
        out_shape=jax.ShapeDtypeStruct((_B, 5), jnp.float32),
        scratch_shapes=[
            pltpu.VMEM((32, _N), jnp.float32),
            pltpu.VMEM((32, _B), jnp.float32),
            pltpu.VMEM((32, _B), jnp.float32),
            pltpu.VMEM((32, _B), jnp.float32),
            pltpu.VMEM((32, _B), jnp.float32),
            pltpu.VMEM((32, _B), jnp.float32),
        ],
    )(cu_seqlens, x_t, norm_mean[:, None], inv[:, None],
      starts[:, None], ends[:, None], starts[None, :], ends[None, :],
      W0, b0[:, None], W1, b1[:, None], W2, b2[:, None],
      W4, b4[None, :], W5, b5[None, :], Wout, bout[None, :])
    return out


# speedup vs baseline: 1.0134x; 1.0134x over previous
import jax
import jax.numpy as jnp
from jax.experimental import pallas as pl
from jax.experimental.pallas import tpu as pltpu

_N = 32768
_B = 16
_H = 2048
_NHALF = 4
_BLK = _H * _NHALF
_NB = _N // _BLK
_HI = jax.lax.Precision.HIGHEST


def _body(cu_ref, x_ref, mean_c_ref, inv_c_ref, sc_ref, ec_ref, sr_ref, er_ref,
          w0t_ref, b0c_ref, w1at_ref, w1bt_ref, b1c_ref, w2t_ref, b2c_ref,
          w4_ref, b4r_ref, w5_ref, b5r_ref, wout_ref, boutr_ref,
          out_ref, segsum, tm, psum, pmn, pmx):
    p = pl.program_id(0)
    i = pl.program_id(1)
    relu = jax.nn.relu

    insegs, onehots, vs = [], [], []
    for h in range(_NHALF):
        idx = (jax.lax.broadcasted_iota(jnp.int32, (_B, _H), 1)
               + i * _BLK + h * _H)
        inseg = (idx >= sc_ref[...]) & (idx < ec_ref[...])
        insegs.append(inseg)
        onehots.append(inseg.astype(jnp.float32))
        x = (x_ref[:, h * _H:(h + 1) * _H] - mean_c_ref[...]) * inv_c_ref[...]
        vs.append(relu(jnp.dot(w0t_ref[...], x) + b0c_ref[...]))

    @pl.when(jnp.logical_and(p == 0, i == 0))
    def _init0():
        segsum[...] = jnp.zeros_like(segsum)

    @pl.when(p == 0)
    def _pass0():
        acc = segsum[...]
        for h in range(_NHALF):
            acc += jax.lax.dot_general(
                vs[h], onehots[h], (((1,), (1,)), ((), ())), precision=_HI)
        segsum[...] = acc

    @pl.when(jnp.logical_and(p == 1, i == 0))
    def _init1():
        cnt = jnp.maximum(er_ref[...] - sr_ref[...], 1).astype(jnp.float32)
        gmean_t = segsum[...] / cnt
        tm[...] = jnp.dot(w1bt_ref[...], gmean_t) + b1c_ref[...]
        psum[...] = jnp.zeros_like(psum)
        pmn[...] = jnp.full_like(pmn, jnp.inf)
        pmx[...] = jnp.full_like(pmx, -jnp.inf)

    @pl.when(p == 1)
    def _pass1():
        v3s = []
        for h in range(_NHALF):
            v2 = relu(jnp.dot(w1at_ref[...], vs[h])
                      + jnp.dot(tm[...], onehots[h], precision=_HI))
            v3s.append(relu(jnp.dot(w2t_ref[...], v2) + b2c_ref[...]))
        acc = psum[...]
        for h in range(_NHALF):
            acc += jax.lax.dot_general(
                v3s[h], onehots[h], (((1,), (1,)), ((), ())), precision=_HI)
        psum[...] = acc
        for h in range(_NHALF):
            base = i * _BLK + h * _H
            for s in range(_B):
                lo = cu_ref[s]
                hi = cu_ref[s + 1]

                @pl.when(jnp.logical_and(hi > base, lo < base + _H))
                def _minmax(s=s, h=h):
                    m = insegs[h][s:s + 1, :]
                    mn = jnp.min(jnp.where(m, v3s[h], jnp.inf),
                                 axis=1, keepdims=True)
                    mx = jnp.max(jnp.where(m, v3s[h], -jnp.inf),
                                 axis=1, keepdims=True)
                    pmn[:, s:s + 1] = jnp.minimum(pmn[:, s:s + 1], mn)
                    pmx[:, s:s + 1] = jnp.maximum(pmx[:, s:s + 1], mx)

    @pl.when(jnp.logical_and(p == 1, i == _NB - 1))
    def _final():
        cntd = er_ref[...] - sr_ref[...]
        cnt = jnp.maximum(cntd, 1).astype(jnp.float32)
        valid = cntd > 0
        pmean_t = psum[...] / cnt
        pmn_t = jnp.where(valid, pmn[...], 0.0)
        pmx_t = jnp.where(valid, pmx[...], 0.0)
        h_t = jnp.concatenate([pmean_t, pmn_t, pmx_t], axis=0)
        h1 = relu(jax.lax.dot_general(
            h_t, w4_ref[...], (((0,), (0,)), ((), ())))
            + b4r_ref[...])
        h2 = relu(jnp.dot(h1, w5_ref[...]) + b5r_ref[...])
        out_ref[...] = jnp.dot(h2, wout_ref[...]) + boutr_ref[...]


def kernel(flat, norm_mean, norm_var, W0, b0, W1, b1, W2, b2, W4, b4, W5, b5,
           Wout, bout, cu_seqlens):
    inv = 1.0 / jnp.sqrt(norm_var)
    w0t = W0.T
    b0c = b0[:, None]
    w1at = W1[:32].T
    w1bt = W1[32:].T
    b1c = b1[:, None]
    w2t = W2.T
    b2c = b2[:, None]
    x_t = flat.T
    starts = cu_seqlens[:-1]
    ends = cu_seqlens[1:]

    def full(shape):
        return pl.BlockSpec(shape, lambda p, i: (0, 0))

    out = pl.pallas_call(
        _body,
        grid=(2, _NB),
        in_specs=[
            pl.BlockSpec(memory_space=pltpu.SMEM),
            pl.BlockSpec((4, _BLK), lambda p, i: (0, i)),
            full((4, 1)), full((4, 1)),
            full((_B, 1)), full((_B, 1)), full((1, _B)), full((1, _B)),
            full((32, 4)), full((32, 1)), full((32, 32)), full((32, 32)),
            full((32, 1)), full((32, 32)), full((32, 1)),
            full((96, 128)), full((1, 128)), full((128, 128)), full((1, 128)),
            full((128, 5)), full((1, 5)),
        ],
        out_specs=full((_B, 5)),
        out_shape=jax.ShapeDtypeStruct((_B, 5), jnp.float32),
        scratch_shapes=[
            pltpu.VMEM((32, _B), jnp.float32),
            pltpu.VMEM((32, _B), jnp.float32),
            pltpu.VMEM((32, _B), jnp.float32),
            pltpu.VMEM((32, _B), jnp.float32),
            pltpu.VMEM((32, _B), jnp.float32),
        ],
    )(cu_seqlens, x_t, norm_mean[:, None], inv[:, None],
      starts[:, None], ends[:, None], starts[None, :], ends[None, :],
      w0t, b0c, w1at, w1bt, b1c, w2t, b2c,
      W4, b4[None, :], W5, b5[None, :], Wout, bout[None, :])
    return out


# speedup vs baseline: 2.9793x; 2.9399x over previous
import jax
import jax.numpy as jnp
from jax.experimental import pallas as pl
from jax.experimental.pallas import tpu as pltpu

_N = 32768
_B = 16
_H = 2048
_NHALF = 2
_BLK = _H * _NHALF
_NB = _N // _BLK
_HI = jax.lax.Precision.HIGHEST


_C00 = (((0,), (0,)), ((), ()))


def _body(cu_ref, x_ref, mean_c_ref, inv_c_ref, sc_ref, ec_ref, sr_ref, er_ref,
          w0_ref, b0c_ref, w1_ref, b1c_ref, w2_ref, b2c_ref,
          w4_ref, b4r_ref, w5_ref, b5r_ref, wout_ref, boutr_ref,
          out_ref, segsum, tm, psum, pmn, pmx):
    p = pl.program_id(0)
    i = pl.program_id(1)
    relu = jax.nn.relu

    insegs, onehots, vs = [], [], []
    for h in range(_NHALF):
        idx = (jax.lax.broadcasted_iota(jnp.int32, (_B, _H), 1)
               + i * _BLK + h * _H)
        inseg = (idx >= sc_ref[...]) & (idx < ec_ref[...])
        insegs.append(inseg)
        onehots.append(inseg.astype(jnp.float32))
        x = (x_ref[:, h * _H:(h + 1) * _H] - mean_c_ref[...]) * inv_c_ref[...]
        vs.append(relu(jax.lax.dot_general(w0_ref[...], x, _C00)
                       + b0c_ref[...]))

    @pl.when(jnp.logical_and(p == 0, i == 0))
    def _init0():
        segsum[...] = jnp.zeros_like(segsum)

    @pl.when(p == 0)
    def _pass0():
        acc = segsum[...]
        for h in range(_NHALF):
            acc += jax.lax.dot_general(
                vs[h], onehots[h], (((1,), (1,)), ((), ())), precision=_HI)
        segsum[...] = acc

    @pl.when(jnp.logical_and(p == 1, i == 0))
    def _init1():
        cnt = jnp.maximum(er_ref[...] - sr_ref[...], 1).astype(jnp.float32)
        gmean_t = segsum[...] / cnt
        tm[...] = (jax.lax.dot_general(w1_ref[32:64, :], gmean_t, _C00)
                   + b1c_ref[...])
        psum[...] = jnp.zeros_like(psum)
        pmn[...] = jnp.full_like(pmn, jnp.inf)
        pmx[...] = jnp.full_like(pmx, -jnp.inf)

    @pl.when(p == 1)
    def _pass1():
        v3s = []
        for h in range(_NHALF):
            v2 = relu(jax.lax.dot_general(w1_ref[0:32, :], vs[h], _C00)
                      + jnp.dot(tm[...], onehots[h], precision=_HI))
            v3s.append(relu(jax.lax.dot_general(w2_ref[...], v2, _C00)
                            + b2c_ref[...]))
        acc = psum[...]
        for h in range(_NHALF):
            acc += jax.lax.dot_general(
                v3s[h], onehots[h], (((1,), (1,)), ((), ())), precision=_HI)
        psum[...] = acc
        for h in range(_NHALF):
            base = i * _BLK + h * _H
            for s in range(_B):
                lo = cu_ref[s]
                hi = cu_ref[s + 1]

                @pl.when(jnp.logical_and(hi > base, lo < base + _H))
                def _minmax(s=s, h=h):
                    m = insegs[h][s:s + 1, :]
                    mn = jnp.min(jnp.where(m, v3s[h], jnp.inf),
                                 axis=1, keepdims=True)
                    mx = jnp.max(jnp.where(m, v3s[h], -jnp.inf),
                                 axis=1, keepdims=True)
                    pmn[:, s:s + 1] = jnp.minimum(pmn[:, s:s + 1], mn)
                    pmx[:, s:s + 1] = jnp.maximum(pmx[:, s:s + 1], mx)

    @pl.when(jnp.logical_and(p == 1, i == _NB - 1))
    def _final():
        cntd = er_ref[...] - sr_ref[...]
        cnt = jnp.maximum(cntd, 1).astype(jnp.float32)
        valid = cntd > 0
        pmean_t = psum[...] / cnt
        pmn_t = jnp.where(valid, pmn[...], 0.0)
        pmx_t = jnp.where(valid, pmx[...], 0.0)
        h_t = jnp.concatenate([pmean_t, pmn_t, pmx_t], axis=0)
        h1 = relu(jax.lax.dot_general(
            h_t, w4_ref[...], (((0,), (0,)), ((), ())))
            + b4r_ref[...])
        h2 = relu(jnp.dot(h1, w5_ref[...]) + b5r_ref[...])
        out_ref[...] = jnp.dot(h2, wout_ref[...]) + boutr_ref[...]


def kernel(flat, norm_mean, norm_var, W0, b0, W1, b1, W2, b2, W4, b4, W5, b5,
           Wout, bout, cu_seqlens):
    inv = 1.0 / jnp.sqrt(norm_var)
    x_t = flat.T
    starts = cu_seqlens[:-1]
    ends = cu_seqlens[1:]

    def full(shape):
        return pl.BlockSpec(shape, lambda p, i: (0, 0))

    out = pl.pallas_call(
        _body,
        grid=(2, _NB),
        in_specs=[
            pl.BlockSpec(memory_space=pltpu.SMEM),
            pl.BlockSpec((4, _BLK), lambda p, i: (0, i)),
            full((4, 1)), full((4, 1)),
            full((_B, 1)), full((_B, 1)), full((1, _B)), full((1, _B)),
            full((4, 32)), full((32, 1)), full((64, 32)), full((32, 1)),
            full((32, 32)), full((32, 1)),
            full((96, 128)), full((1, 128)), full((128, 128)), full((1, 128)),
            full((128, 5)), full((1, 5)),
        ],
        out_specs=full((_B, 5)),
        out_shape=jax.ShapeDtypeStruct((_B, 5), jnp.float32),
        scratch_shapes=[
            pltpu.VMEM((32, _B), jnp.float32),
            pltpu.VMEM((32, _B), jnp.float32),
            pltpu.VMEM((32, _B), jnp.float32),
            pltpu.VMEM((32, _B), jnp.float32),
            pltpu.VMEM((32, _B), jnp.float32),
        ],
    )(cu_seqlens, x_t, norm_mean[:, None], inv[:, None],
      starts[:, None], ends[:, None], starts[None, :], ends[None, :],
      W0, b0[:, None], W1, b1[:, None], W2, b2[:, None],
      W4, b4[None, :], W5, b5[None, :], Wout, bout[None, :])
    return out
